# baseline (speedup 1.0000x reference)
import functools

import jax
import jax.numpy as jnp
import numpy as np
from jax import lax
from jax.experimental import pallas as pl
from jax.experimental.pallas import tpu as pltpu
from jax.experimental.pallas import tpu_sc as plsc

_N = 2048
_M = _N * (_N - 1) // 2
_CHUNK = _N + 8
_Q = 512
_NC = 2
_NS = 16
_NW = _NC * _NS
_RPW = _N // _NW
_DEPTH = 4
_B = 256
_NB = _N // _B
_NPAIR = _NB * (_NB + 1) // 2


def _phase1_body(pp_hbm, u_hbm, ins, obs, sis, sos):
    wid = lax.axis_index("s") * _NC + lax.axis_index("c")

    def row_params(t):
        i = t * _NW + wid
        off = i * (_N - 1) - (i * (i - 1)) // 2
        start = off - i - 1
        a8 = pl.multiple_of(
            jnp.clip((start // 8) * 8, 0, _M - _CHUNK).astype(jnp.int32), 8)
        r = jnp.where(i == 0, 7, start - a8)
        return i, a8, r

    def in_copy(t, s, q):
        _, a8, _ = row_params(t)
        sz = _N - q * _Q + 8
        return pltpu.make_async_copy(
            pp_hbm.at[pl.ds(a8 + q * _Q, sz)],
            ins[s].at[pl.ds(q * _Q, sz)], sis[s])

    def in_row0(t, s):
        return pltpu.make_async_copy(
            pp_hbm.at[pl.ds(0, _N)], ins[s].at[pl.ds(8, _N)], sis[s])

    def out_copy(t, s, q):
        i, _, _ = row_params(t)
        sz = _N - q * _Q
        return pltpu.make_async_copy(
            obs[s].at[pl.ds(q * _Q, sz)],
            u_hbm.at[i, pl.ds(q * _Q, sz)], sos[s])

    def _dispatch_in(t, s, act):
        i, _, _ = row_params(t)
        iq = i // _Q
        for q in range(_N // _Q):
            cond = jnp.logical_and(iq == q, i > 0) if q == 0 else iq == q

            @pl.when(cond)
            def _():
                act(in_copy(t, s, q))

        @pl.when(i == 0)
        def _():
            act(in_row0(t, s))

    def _dispatch_out(t, s, act):
        i, _, _ = row_params(t)
        iq = i // _Q
        for q in range(_N // _Q):
            @pl.when(iq == q)
            def _():
                act(out_copy(t, s, q))

    def in_start(t, s):
        _dispatch_in(t, s, lambda c: c.start())

    def in_wait(t, s):
        _dispatch_in(t, s, lambda c: c.wait())

    def out_start(t, s):
        _dispatch_out(t, s, lambda c: c.start())

    def out_wait(t, s):
        _dispatch_out(t, s, lambda c: c.wait())

    def shift(t, s):
        i, _, r = row_params(t)
        ib = ins[s]
        ob = obs[s]
        kb0 = (i + 1) // 256

        def blk(kb, c2):
            base = kb * 256
            vals = [ib[pl.ds(r + base + u * 16, 16)] for u in range(16)]
            for u, v in enumerate(vals):
                ob[pl.ds(base + u * 16, 16)] = v
            return c2

        lax.fori_loop(kb0, _N // 256, blk, 0)

    for t in range(_DEPTH - 1):
        in_start(t, t)

    def body(t4, carry):
        for s in range(_DEPTH):
            t = _DEPTH * t4 + s

            @pl.when(t + _DEPTH - 1 < _RPW)
            def _():
                in_start(t + _DEPTH - 1, (s + _DEPTH - 1) % _DEPTH)

            in_wait(t, s)

            @pl.when(t >= _DEPTH)
            def _():
                out_wait(t - _DEPTH, s)

            shift(t, s)
            out_start(t, s)
        return carry

    lax.fori_loop(0, _RPW // _DEPTH, body, 0)
    for t in range(_RPW - _DEPTH, _RPW):
        out_wait(t, t % _DEPTH)


@functools.partial(
    pl.kernel,
    out_type=jax.ShapeDtypeStruct((_N, _N), jnp.float32),
    mesh=plsc.VectorSubcoreMesh(core_axis_name="c", subcore_axis_name="s"),
    scratch_types=(
        [pltpu.VMEM((_CHUNK,), jnp.float32)] * _DEPTH
        + [pltpu.VMEM((_N,), jnp.float32)] * _DEPTH
        + [pltpu.SemaphoreType.DMA] * (2 * _DEPTH)
    ),
)
def _phase1(pp_hbm, u_hbm, *bufs):
    ins = bufs[0:_DEPTH]
    obs = bufs[_DEPTH:2 * _DEPTH]
    sis = bufs[2 * _DEPTH:3 * _DEPTH]
    sos = bufs[3 * _DEPTH:4 * _DEPTH]
    _phase1_body(pp_hbm, u_hbm, ins, obs, sis, sos)


def _phase2_body(bi_ref, bj_ref, a_ref, o_ref):
    g = pl.program_id(0)
    bi = bi_ref[g]
    bj = bj_ref[g]
    ua = a_ref[...]
    ir = lax.broadcasted_iota(jnp.int32, (_B, _B), 0)
    ic = lax.broadcasted_iota(jnp.int32, (_B, _B), 1)
    zero = jnp.float32(0.0)
    gr = ir + bj * _B
    gc = ic + bi * _B
    uat = ua.T
    o_ref[...] = jnp.where(gc < gr, -uat, jnp.where(gc > gr, ua, zero))


def _phase2(a):
    pairs = [(x, y) for x in range(_NB) for y in range(x, _NB)]
    bi = jnp.asarray(np.array([p[0] for p in pairs], dtype=np.int32))
    bj = jnp.asarray(np.array([p[1] for p in pairs], dtype=np.int32))
    grid_spec = pltpu.PrefetchScalarGridSpec(
        num_scalar_prefetch=2,
        grid=(_NPAIR,),
        in_specs=[
            pl.BlockSpec((_B, _B), lambda g, bi, bj: (bi[g], bj[g])),
        ],
        out_specs=pl.BlockSpec((_B, _B), lambda g, bi, bj: (bj[g], bi[g])),
    )
    return pl.pallas_call(
        _phase2_body,
        grid_spec=grid_spec,
        out_shape=jax.ShapeDtypeStruct((_N, _N), jnp.float32),
        input_output_aliases={2: 0},
    )(bi, bj, a)


def kernel(params, triu_indices):
    del triu_indices
    a = _phase1(params.astype(jnp.float32))
    return _phase2(a)


# speedup vs baseline: 1.2348x; 1.2348x over previous
import functools

import jax
import jax.numpy as jnp
import numpy as np
from jax import lax
from jax.experimental import pallas as pl
from jax.experimental.pallas import tpu as pltpu
from jax.experimental.pallas import tpu_sc as plsc

_N = 2048
_M = _N * (_N - 1) // 2
_CHUNK = _N + 8
_Q = 512
_NC = 2
_NS = 16
_NW = _NC * _NS
_RPW = _N // _NW
_DEPTH = 4
_B = 512
_NB = _N // _B
_NPAIR = _NB * (_NB + 1) // 2


def _phase1_body(pp_hbm, a_hbm, ins, obs, sis, sos):
    wid = lax.axis_index("s") * _NC + lax.axis_index("c")

    def row_params(t):
        i = t * _NW + wid
        off = i * (_N - 1) - (i * (i - 1)) // 2
        start = off - i - 1
        a8 = pl.multiple_of(
            jnp.clip((start // 8) * 8, 0, _M - _CHUNK).astype(jnp.int32), 8)
        r = jnp.where(i == 0, 7, start - a8)
        return i, a8, r

    def in_copy(t, s, q):
        _, a8, _ = row_params(t)
        sz = _N - q * _Q + 8
        return pltpu.make_async_copy(
            pp_hbm.at[pl.ds(a8 + q * _Q, sz)],
            ins[s].at[pl.ds(q * _Q, sz)], sis[s])

    def in_row0(t, s):
        return pltpu.make_async_copy(
            pp_hbm.at[pl.ds(0, _N)], ins[s].at[pl.ds(8, _N)], sis[s])

    def out_copy(t, s, q):
        i, _, _ = row_params(t)
        sz = _N - q * _Q
        return pltpu.make_async_copy(
            obs[s].at[pl.ds(q * _Q, sz)],
            a_hbm.at[i, pl.ds(q * _Q, sz)], sos[s])

    def _dispatch_in(t, s, act):
        i, _, _ = row_params(t)
        iq = i // _Q
        for q in range(_N // _Q):
            cond = jnp.logical_and(iq == q, i > 0) if q == 0 else iq == q

            @pl.when(cond)
            def _():
                act(in_copy(t, s, q))

        @pl.when(i == 0)
        def _():
            act(in_row0(t, s))

    def _dispatch_out(t, s, act):
        i, _, _ = row_params(t)
        iq = i // _Q
        for q in range(_N // _Q):
            @pl.when(iq == q)
            def _():
                act(out_copy(t, s, q))

    def in_start(t, s):
        _dispatch_in(t, s, lambda c: c.start())

    def in_wait(t, s):
        _dispatch_in(t, s, lambda c: c.wait())

    def out_start(t, s):
        _dispatch_out(t, s, lambda c: c.start())

    def out_wait(t, s):
        _dispatch_out(t, s, lambda c: c.wait())

    def shift(t, s):
        i, _, r = row_params(t)
        ib = ins[s]
        ob = obs[s]
        kb0 = (i + 1) // 256

        def blk(kb, c2):
            base = kb * 256
            vals = [ib[pl.ds(r + base + u * 16, 16)] for u in range(16)]
            for u, v in enumerate(vals):
                ob[pl.ds(base + u * 16, 16)] = v
            return c2

        lax.fori_loop(kb0, _N // 256, blk, 0)

    for t in range(_DEPTH - 1):
        in_start(t, t)

    def body(t4, carry):
        for s in range(_DEPTH):
            t = _DEPTH * t4 + s

            @pl.when(t + _DEPTH - 1 < _RPW)
            def _():
                in_start(t + _DEPTH - 1, (s + _DEPTH - 1) % _DEPTH)

            in_wait(t, s)

            @pl.when(t >= _DEPTH)
            def _():
                out_wait(t - _DEPTH, s)

            shift(t, s)
            out_start(t, s)
        return carry

    lax.fori_loop(0, _RPW // _DEPTH, body, 0)
    for t in range(_RPW - _DEPTH, _RPW):
        out_wait(t, t % _DEPTH)


@functools.partial(
    pl.kernel,
    out_type=jax.ShapeDtypeStruct((_N, _N), jnp.float32),
    mesh=plsc.VectorSubcoreMesh(core_axis_name="c", subcore_axis_name="s"),
    scratch_types=(
        [pltpu.VMEM((_CHUNK,), jnp.float32)] * _DEPTH
        + [pltpu.VMEM((_N,), jnp.float32)] * _DEPTH
        + [pltpu.SemaphoreType.DMA] * (2 * _DEPTH)
    ),
)
def _phase1(pp_hbm, a_hbm, *bufs):
    ins = bufs[0:_DEPTH]
    obs = bufs[_DEPTH:2 * _DEPTH]
    sis = bufs[2 * _DEPTH:3 * _DEPTH]
    sos = bufs[3 * _DEPTH:4 * _DEPTH]
    _phase1_body(pp_hbm, a_hbm, ins, obs, sis, sos)


def _phase2_body(bi_ref, bj_ref, a_ref, o_ref):
    g = pl.program_id(0)
    bi = bi_ref[g]
    bj = bj_ref[g]
    ua = a_ref[...]
    ir = lax.broadcasted_iota(jnp.int32, (_B, _B), 0)
    ic = lax.broadcasted_iota(jnp.int32, (_B, _B), 1)
    zero = jnp.float32(0.0)
    gr = ir + bj * _B
    gc = ic + bi * _B
    uat = ua.T
    o_ref[...] = jnp.where(gc < gr, -uat, jnp.where(gc > gr, ua, zero))


def _phase2(a):
    pairs = [(x, y) for x in range(_NB) for y in range(x, _NB)]
    bi = jnp.asarray(np.array([p[0] for p in pairs], dtype=np.int32))
    bj = jnp.asarray(np.array([p[1] for p in pairs], dtype=np.int32))
    grid_spec = pltpu.PrefetchScalarGridSpec(
        num_scalar_prefetch=2,
        grid=(_NPAIR,),
        in_specs=[
            pl.BlockSpec((_B, _B), lambda g, bi, bj: (bi[g], bj[g])),
        ],
        out_specs=pl.BlockSpec((_B, _B), lambda g, bi, bj: (bj[g], bi[g])),
    )
    return pl.pallas_call(
        _phase2_body,
        grid_spec=grid_spec,
        out_shape=jax.ShapeDtypeStruct((_N, _N), jnp.float32),
        input_output_aliases={2: 0},
    )(bi, bj, a)


def kernel(params, triu_indices):
    del triu_indices
    a = _phase1(params.astype(jnp.float32))
    return _phase2(a)
